# Initial kernel scaffold; baseline (speedup 1.0000x reference)
import jax
import jax.numpy as jnp
from jax.experimental import pallas as pl


def kernel(diffusion_step, embedding, W1, b1, W2, b2):
    raise NotImplementedError("write your pallas kernel here")


# speedup vs baseline: 1.6329x; 1.6329x over previous
import functools

import jax
import jax.numpy as jnp
from jax import lax
from jax.experimental import pallas as pl
from jax.experimental.pallas import tpu as pltpu
from jax.experimental.pallas import tpu_sc as plsc

MAX_STEPS = 1000
TBL = 1024
IN_DIM = 256
D = 1024
B = 16384

NC = 2
NS = 16
NW = NC * NS
B_PER_W = B // NW
CHUNK = 64
NCH = B_PER_W // CHUNK


def _sigmoid(x):
    return 1.0 / (1.0 + jnp.exp(-x))


def _mlp_table_body(e_ref, w1_ref, b1_ref, w2_ref, b2_ref, y_ref):
    h = jnp.dot(e_ref[...], w1_ref[...], preferred_element_type=jnp.float32)
    h = h + b1_ref[...]
    h = h * _sigmoid(h)
    y = jnp.dot(h, w2_ref[...], preferred_element_type=jnp.float32)
    y = y + b2_ref[...]
    y_ref[...] = y * _sigmoid(y)


def _mlp_table(e_pad, W1, b1, W2, b2):
    return pl.pallas_call(
        _mlp_table_body,
        out_shape=jax.ShapeDtypeStruct((TBL, D), jnp.float32),
    )(e_pad, W1, b1.reshape(1, D), W2, b2.reshape(1, D))


_sc_mesh = plsc.VectorSubcoreMesh(core_axis_name="c", subcore_axis_name="s")


@functools.partial(
    pl.kernel,
    out_type=jax.ShapeDtypeStruct((B, D), jnp.float32),
    mesh=_sc_mesh,
    scratch_types=[
        pltpu.VMEM((NCH, CHUNK), jnp.int32),
        pltpu.VMEM((CHUNK, D), jnp.float32),
        pltpu.SemaphoreType.DMA,
    ],
)
def _sc_gather(table_hbm, idx_hbm, out_hbm, idx_v, rows_v, sem):
    wid = lax.axis_index("s") * NC + lax.axis_index("c")
    base = wid * B_PER_W
    pltpu.sync_copy(idx_hbm.at[wid], idx_v)
    for c in range(NCH):
        pltpu.async_copy(table_hbm.at[idx_v.at[c]], rows_v, sem).wait()
        pltpu.sync_copy(rows_v, out_hbm.at[pl.ds(base + c * CHUNK, CHUNK)])


def kernel(diffusion_step, embedding, W1, b1, W2, b2):
    e_pad = jnp.zeros((TBL, IN_DIM), jnp.float32).at[:MAX_STEPS].set(embedding)
    y = _mlp_table(e_pad, W1, b1, W2, b2)
    idx = diffusion_step.reshape(NW, NCH, CHUNK)
    return _sc_gather(y, idx)


# speedup vs baseline: 1.6627x; 1.0182x over previous
import functools

import jax
import jax.numpy as jnp
from jax import lax
from jax.experimental import pallas as pl
from jax.experimental.pallas import tpu as pltpu
from jax.experimental.pallas import tpu_sc as plsc

TBL = 1000
IN_DIM = 256
D = 1024
B = 16384

NC = 2
NS = 16
NW = NC * NS
B_PER_W = B // NW
CHUNK = 32
NCH = B_PER_W // CHUNK


def _sigmoid(x):
    return 1.0 / (1.0 + jnp.exp(-x))


def _mlp_table_body(e_ref, w1_ref, b1_ref, w2_ref, b2_ref, y_ref):
    h = jnp.dot(e_ref[...], w1_ref[...], preferred_element_type=jnp.float32)
    h = h + b1_ref[...]
    h = h * _sigmoid(h)
    y = jnp.dot(h, w2_ref[...], preferred_element_type=jnp.float32)
    y = y + b2_ref[...]
    y_ref[...] = y * _sigmoid(y)


def _mlp_table(e_pad, W1, b1, W2, b2):
    return pl.pallas_call(
        _mlp_table_body,
        out_shape=jax.ShapeDtypeStruct((TBL, D), jnp.float32),
    )(e_pad, W1, b1.reshape(1, D), W2, b2.reshape(1, D))


_sc_mesh = plsc.VectorSubcoreMesh(core_axis_name="c", subcore_axis_name="s")


@functools.partial(
    pl.kernel,
    out_type=jax.ShapeDtypeStruct((B, D), jnp.float32),
    mesh=_sc_mesh,
    scratch_types=[
        pltpu.VMEM((NCH, CHUNK), jnp.int32),
        pltpu.VMEM((CHUNK, D), jnp.float32),
        pltpu.VMEM((CHUNK, D), jnp.float32),
        pltpu.SemaphoreType.DMA,
        pltpu.SemaphoreType.DMA,
        pltpu.SemaphoreType.DMA,
        pltpu.SemaphoreType.DMA,
    ],
)
def _sc_gather(table_hbm, idx_hbm, out_hbm, idx_v, buf0, buf1, g0, g1, p0, p1):
    wid = lax.axis_index("s") * NC + lax.axis_index("c")
    base = wid * B_PER_W
    bufs = (buf0, buf1)
    gsem = (g0, g1)
    psem = (p0, p1)
    pltpu.sync_copy(idx_hbm.at[wid], idx_v)
    gets = [None, None]
    puts = [None, None]
    gets[0] = pltpu.async_copy(table_hbm.at[idx_v.at[0]], bufs[0], gsem[0])
    for c in range(NCH):
        cur = c % 2
        nxt = (c + 1) % 2
        gets[cur].wait()
        if c + 1 < NCH:
            if puts[nxt] is not None:
                puts[nxt].wait()
            gets[nxt] = pltpu.async_copy(
                table_hbm.at[idx_v.at[c + 1]], bufs[nxt], gsem[nxt])
        puts[cur] = pltpu.async_copy(
            bufs[cur], out_hbm.at[pl.ds(base + c * CHUNK, CHUNK)], psem[cur])
    puts[(NCH - 2) % 2].wait()
    puts[(NCH - 1) % 2].wait()


def kernel(diffusion_step, embedding, W1, b1, W2, b2):
    y = _mlp_table(embedding, W1, b1, W2, b2)
    idx = diffusion_step.reshape(NW, NCH, CHUNK)
    return _sc_gather(y, idx)
